# Initial kernel scaffold; baseline (speedup 1.0000x reference)
import jax
import jax.numpy as jnp
from jax.experimental import pallas as pl


def kernel(edge_index, adj_vals, users, items, emb_user, emb_item):
    raise NotImplementedError("write your pallas kernel here")


# speedup vs baseline: 2.7426x; 2.7426x over previous
import functools
import jax
import jax.numpy as jnp
from jax import lax
from jax.experimental import pallas as pl
from jax.experimental.pallas import tpu as pltpu
from jax.experimental.pallas import tpu_sc as plsc

_N_USER = 20000
_N_ITEM = 30000
_N = _N_USER + _N_ITEM
_E = 800000
_D = 64
_B = 4096
_NC, _NS = 2, 16
_RPC = 25600
_SPR = _RPC // _NS
_WCH = 80
_EPS = _E // _NS
_EC = 400
_ET = 80
_NT = _EC // _ET
_NCH = _EPS // _EC
_BPW = _B // (_NC * _NS)

_mesh = plsc.VectorSubcoreMesh(core_axis_name="c", subcore_axis_name="s")


def _layer_body(row_h, col_h, val_h, xsrc_h, xdst_h,
                acc, rbuf, cbuf, vbuf, lidx, rows, sem):
  c = lax.axis_index("c")
  s = lax.axis_index("s")
  base = c * _RPC
  zero16 = jnp.zeros((16,), jnp.float32)

  def zrow(i, carry):
    for k in range(_D // 16):
      rows[i, pl.ds(k * 16, 16)] = zero16
    return carry
  lax.fori_loop(0, _WCH, zrow, 0)

  def zacc(i, carry):
    pltpu.sync_copy(rows.at[pl.ds(0, _WCH)],
                    acc.at[pl.ds(s * _SPR + i * _WCH, _WCH)])
    return carry
  lax.fori_loop(0, _SPR // _WCH, zacc, 0)

  plsc.subcore_barrier()

  e_base = s * _EPS

  def echunk(i, carry):
    e0 = e_base + i * _EC
    pltpu.sync_copy(row_h.at[pl.ds(e0, _EC)], rbuf)
    pltpu.sync_copy(col_h.at[pl.ds(e0, _EC)], cbuf)
    pltpu.sync_copy(val_h.at[pl.ds(e0, _EC)], vbuf)
    for j in range(_EC // 16):
      t, off = (j * 16) // _ET, (j * 16) % _ET
      r = rbuf[pl.ds(j * 16, 16)]
      li = r - base
      owned = (li >= 0) & (li < _RPC)
      lidx[t, pl.ds(off, 16)] = jnp.where(owned, li, 0)
      v = vbuf[pl.ds(j * 16, 16)]
      vbuf[pl.ds(j * 16, 16)] = jnp.where(owned, v, jnp.float32(0.0))
    cps = [
        pltpu.async_copy(xsrc_h.at[cbuf.at[pl.ds(t * _ET, _ET)]],
                         rows.at[pl.ds(t * _ET, _ET)], sem)
        for t in range(_NT)
    ]
    for cp in cps:
      cp.wait()
    def scale(j, carry):
      vv = vbuf[pl.ds(j * 16, 16)]
      for l in range(16):
        e = j * 16 + l
        vsp = jnp.full((16,), vv[l])
        for k in range(_D // 16):
          rows[e, pl.ds(k * 16, 16)] = rows[e, pl.ds(k * 16, 16)] * vsp
      return carry
    lax.fori_loop(0, _EC // 16, scale, 0)
    for t in range(_NT):
      pltpu.sync_copy(rows.at[pl.ds(t * _ET, _ET)],
                      acc.at[lidx.at[t]], add=True)
    return carry
  lax.fori_loop(0, _NCH, echunk, 0)

  plsc.subcore_barrier()

  g0 = base + s * _SPR
  nch = jnp.minimum(_SPR, jnp.maximum(0, _N - g0)) // _WCH

  def wback(i, carry):
    pltpu.sync_copy(acc.at[pl.ds(s * _SPR + i * _WCH, _WCH)],
                    xdst_h.at[pl.ds(g0 + i * _WCH, _WCH)])
    return carry
  lax.fori_loop(0, nch, wback, 0)


_layer = functools.partial(
    pl.kernel,
    out_type=jax.ShapeDtypeStruct((_N, _D), jnp.float32),
    scratch_types=[
        pltpu.VMEM_SHARED((_RPC, _D), jnp.float32),
        pltpu.VMEM((_EC,), jnp.int32),
        pltpu.VMEM((_EC,), jnp.int32),
        pltpu.VMEM((_EC,), jnp.float32),
        pltpu.VMEM((_NT, _ET), jnp.int32),
        pltpu.VMEM((_EC, _D), jnp.float32),
        pltpu.SemaphoreType.DMA,
    ],
    mesh=_mesh,
    compiler_params=pltpu.CompilerParams(use_tc_tiling_on_sc=False, needs_layout_passes=False),
)(_layer_body)


def _final_body(x0_h, x1_h, x2_h, x3_h, users_h, items_h, gamma_h,
                ubuf, ibuf, usum, isum, tbuf, gbuf, sem):
  c = lax.axis_index("c")
  s = lax.axis_index("s")
  wid = s * _NC + c
  b0 = wid * _BPW
  pltpu.sync_copy(users_h.at[pl.ds(b0, _BPW)], ubuf)
  pltpu.sync_copy(items_h.at[pl.ds(b0, _BPW)], ibuf)
  for j in range(_BPW // 16):
    ibuf[pl.ds(j * 16, 16)] = ibuf[pl.ds(j * 16, 16)] + _N_USER

  pltpu.async_copy(x0_h.at[ubuf], usum, sem).wait()
  pltpu.async_copy(x0_h.at[ibuf], isum, sem).wait()
  for xl_h in (x1_h, x2_h, x3_h):
    pltpu.async_copy(xl_h.at[ubuf], tbuf, sem).wait()

    def accu(i, carry):
      for k in range(_D // 16):
        sl = pl.ds(k * 16, 16)
        usum[i, sl] = usum[i, sl] + tbuf[i, sl]
      return carry
    lax.fori_loop(0, _BPW, accu, 0)

    pltpu.async_copy(xl_h.at[ibuf], tbuf, sem).wait()

    def acci(i, carry):
      for k in range(_D // 16):
        sl = pl.ds(k * 16, 16)
        isum[i, sl] = isum[i, sl] + tbuf[i, sl]
      return carry
    lax.fori_loop(0, _BPW, acci, 0)

  lane = lax.iota(jnp.int32, 16)

  def group(g, carry):
    out16 = jnp.zeros((16,), jnp.float32)
    for l in range(16):
      i = g * 16 + l
      acc2 = jnp.zeros((16,), jnp.float32)
      for k in range(_D // 16):
        sl = pl.ds(k * 16, 16)
        acc2 = acc2 + usum[i, sl] * isum[i, sl]
      dot = jnp.sum(acc2) * jnp.float32(0.0625)
      out16 = jnp.where(lane == l, jnp.full((16,), dot), out16)
    gbuf[pl.ds(g * 16, 16)] = out16
    return carry
  lax.fori_loop(0, _BPW // 16, group, 0)

  pltpu.sync_copy(gbuf, gamma_h.at[pl.ds(b0, _BPW)])


_final = functools.partial(
    pl.kernel,
    out_type=jax.ShapeDtypeStruct((_B,), jnp.float32),
    scratch_types=[
        pltpu.VMEM((_BPW,), jnp.int32),
        pltpu.VMEM((_BPW,), jnp.int32),
        pltpu.VMEM((_BPW, _D), jnp.float32),
        pltpu.VMEM((_BPW, _D), jnp.float32),
        pltpu.VMEM((_BPW, _D), jnp.float32),
        pltpu.VMEM((_BPW,), jnp.float32),
        pltpu.SemaphoreType.DMA,
    ],
    mesh=_mesh,
    compiler_params=pltpu.CompilerParams(use_tc_tiling_on_sc=False, needs_layout_passes=False),
)(_final_body)


def kernel(edge_index, adj_vals, users, items, emb_user, emb_item):
  row = edge_index[0]
  col = edge_index[1]
  x0 = jnp.concatenate([emb_user, emb_item], axis=0)
  x1 = _layer(row, col, adj_vals, x0)
  x2 = _layer(row, col, adj_vals, x1)
  x3 = _layer(row, col, adj_vals, x2)
  return _final(x0, x1, x2, x3, users, items)


# speedup vs baseline: 5.9150x; 2.1567x over previous
import functools
import jax
import jax.numpy as jnp
from jax import lax
from jax.experimental import pallas as pl
from jax.experimental.pallas import tpu as pltpu
from jax.experimental.pallas import tpu_sc as plsc

_N_USER = 20000
_N_ITEM = 30000
_N = _N_USER + _N_ITEM
_E = 800000
_D = 64
_B = 4096
_NC, _NS = 2, 16
_RPC = 25600
_SPR = _RPC // _NS
_WCH = 80
_EPS = _E // _NS
_EC = 400
_ET = 80
_NT = _EC // _ET
_NCH = _EPS // _EC
_BPW = _B // (_NC * _NS)

_mesh = plsc.VectorSubcoreMesh(core_axis_name="c", subcore_axis_name="s")


def _layer_body(row_h, col_h, val_h, xsrc_h, xdst_h,
                acc, rbuf, cbuf, vbuf, lidx, rows, gsem, esem):
  c = lax.axis_index("c")
  s = lax.axis_index("s")
  base = c * _RPC
  zero16 = jnp.zeros((16,), jnp.float32)

  def zrow(i, carry):
    for k in range(_D // 16):
      rows[0, i, pl.ds(k * 16, 16)] = zero16
    return carry
  lax.fori_loop(0, _WCH, zrow, 0)

  def zacc(i, carry):
    pltpu.sync_copy(rows.at[0],
                    acc.at[pl.ds(s * _SPR + i * _WCH, _WCH)])
    return carry
  lax.fori_loop(0, _SPR // _WCH, zacc, 0)

  plsc.subcore_barrier()

  e_base = s * _EPS

  def load_group(g, p):
    e0 = e_base + g * _EC
    pltpu.async_copy(row_h.at[pl.ds(e0, _EC)], rbuf.at[p], esem)
    pltpu.async_copy(col_h.at[pl.ds(e0, _EC)], cbuf.at[p], esem)
    pltpu.async_copy(val_h.at[pl.ds(e0, _EC)], vbuf.at[p], esem)

  def wait_group(p):
    pltpu.make_async_copy(row_h.at[pl.ds(0, _EC)], rbuf.at[p], esem).wait()
    pltpu.make_async_copy(col_h.at[pl.ds(0, _EC)], cbuf.at[p], esem).wait()
    pltpu.make_async_copy(val_h.at[pl.ds(0, _EC)], vbuf.at[p], esem).wait()

  def mask_group(p):
    for j in range(_EC // 16):
      t, off = (j * 16) // _ET, (j * 16) % _ET
      r = rbuf[p, pl.ds(j * 16, 16)]
      li = r - base
      owned = (li >= 0) & (li < _RPC)
      lidx[p, t, pl.ds(off, 16)] = jnp.where(owned, li, 0)
      v = vbuf[p, pl.ds(j * 16, 16)]
      vbuf[p, pl.ds(j * 16, 16)] = jnp.where(owned, v, jnp.float32(0.0))

  def fire_gather(p, t, b):
    pltpu.async_copy(xsrc_h.at[cbuf.at[p, pl.ds(t * _ET, _ET)]],
                     rows.at[b], gsem)

  def wait_gather(b):
    pltpu.make_async_copy(xsrc_h.at[pl.ds(0, _ET)], rows.at[b], gsem).wait()

  load_group(0, 0)
  wait_group(0)
  load_group(1, 1)
  mask_group(0)
  fire_gather(0, 0, 0)

  def group_body(g, carry):
    gp = lax.rem(g, 2)
    for t in range(_NT):
      b = lax.rem(g + t, 2)
      wait_gather(b)
      if t < _NT - 1:
        fire_gather(gp, t + 1, 1 - b)
      def scale(j, carry2):
        vv = vbuf[gp, pl.ds(t * _ET + j * 16, 16)]
        for l in range(16):
          e = j * 16 + l
          vsp = jnp.full((16,), vv[l])
          for k in range(_D // 16):
            rows[b, e, pl.ds(k * 16, 16)] = (
                rows[b, e, pl.ds(k * 16, 16)] * vsp)
        return carry2
      lax.fori_loop(0, _ET // 16, scale, 0)
      pltpu.sync_copy(rows.at[b], acc.at[lidx.at[gp, t]], add=True)
    gn = g + 1
    gnp = 1 - gp

    @pl.when(gn < _NCH)
    def _boundary():
      wait_group(gnp)

      @pl.when(gn + 1 < _NCH)
      def _prefetch():
        load_group(gn + 1, gp)
      mask_group(gnp)
      fire_gather(gnp, 0, lax.rem(gn, 2))
    return carry
  lax.fori_loop(0, _NCH, group_body, 0)

  plsc.subcore_barrier()

  g0 = base + s * _SPR
  nch = jnp.minimum(_SPR, jnp.maximum(0, _N - g0)) // _WCH

  def wback(i, carry):
    pltpu.sync_copy(acc.at[pl.ds(s * _SPR + i * _WCH, _WCH)],
                    xdst_h.at[pl.ds(g0 + i * _WCH, _WCH)])
    return carry
  lax.fori_loop(0, nch, wback, 0)


_layer = functools.partial(
    pl.kernel,
    out_type=jax.ShapeDtypeStruct((_N, _D), jnp.float32),
    scratch_types=[
        pltpu.VMEM_SHARED((_RPC, _D), jnp.float32),
        pltpu.VMEM((2, _EC), jnp.int32),
        pltpu.VMEM((2, _EC), jnp.int32),
        pltpu.VMEM((2, _EC), jnp.float32),
        pltpu.VMEM((2, _NT, _ET), jnp.int32),
        pltpu.VMEM((2, _ET, _D), jnp.float32),
        pltpu.SemaphoreType.DMA,
        pltpu.SemaphoreType.DMA,
    ],
    mesh=_mesh,
    compiler_params=pltpu.CompilerParams(use_tc_tiling_on_sc=False, needs_layout_passes=False),
)(_layer_body)


def _final_body(x0_h, x1_h, x2_h, x3_h, users_h, items_h, gamma_h,
                ubuf, ibuf, usum, isum, tbuf, gbuf, sem):
  c = lax.axis_index("c")
  s = lax.axis_index("s")
  wid = s * _NC + c
  b0 = wid * _BPW
  pltpu.sync_copy(users_h.at[pl.ds(b0, _BPW)], ubuf)
  pltpu.sync_copy(items_h.at[pl.ds(b0, _BPW)], ibuf)
  for j in range(_BPW // 16):
    ibuf[pl.ds(j * 16, 16)] = ibuf[pl.ds(j * 16, 16)] + _N_USER

  pltpu.async_copy(x0_h.at[ubuf], usum, sem).wait()
  pltpu.async_copy(x0_h.at[ibuf], isum, sem).wait()
  for xl_h in (x1_h, x2_h, x3_h):
    pltpu.async_copy(xl_h.at[ubuf], tbuf, sem).wait()

    def accu(i, carry):
      for k in range(_D // 16):
        sl = pl.ds(k * 16, 16)
        usum[i, sl] = usum[i, sl] + tbuf[i, sl]
      return carry
    lax.fori_loop(0, _BPW, accu, 0)

    pltpu.async_copy(xl_h.at[ibuf], tbuf, sem).wait()

    def acci(i, carry):
      for k in range(_D // 16):
        sl = pl.ds(k * 16, 16)
        isum[i, sl] = isum[i, sl] + tbuf[i, sl]
      return carry
    lax.fori_loop(0, _BPW, acci, 0)

  lane = lax.iota(jnp.int32, 16)

  def group(g, carry):
    out16 = jnp.zeros((16,), jnp.float32)
    for l in range(16):
      i = g * 16 + l
      acc2 = jnp.zeros((16,), jnp.float32)
      for k in range(_D // 16):
        sl = pl.ds(k * 16, 16)
        acc2 = acc2 + usum[i, sl] * isum[i, sl]
      dot = jnp.sum(acc2) * jnp.float32(0.0625)
      out16 = jnp.where(lane == l, jnp.full((16,), dot), out16)
    gbuf[pl.ds(g * 16, 16)] = out16
    return carry
  lax.fori_loop(0, _BPW // 16, group, 0)

  pltpu.sync_copy(gbuf, gamma_h.at[pl.ds(b0, _BPW)])


_final = functools.partial(
    pl.kernel,
    out_type=jax.ShapeDtypeStruct((_B,), jnp.float32),
    scratch_types=[
        pltpu.VMEM((_BPW,), jnp.int32),
        pltpu.VMEM((_BPW,), jnp.int32),
        pltpu.VMEM((_BPW, _D), jnp.float32),
        pltpu.VMEM((_BPW, _D), jnp.float32),
        pltpu.VMEM((_BPW, _D), jnp.float32),
        pltpu.VMEM((_BPW,), jnp.float32),
        pltpu.SemaphoreType.DMA,
    ],
    mesh=_mesh,
    compiler_params=pltpu.CompilerParams(use_tc_tiling_on_sc=False, needs_layout_passes=False),
)(_final_body)


def kernel(edge_index, adj_vals, users, items, emb_user, emb_item):
  row = edge_index[0]
  col = edge_index[1]
  x0 = jnp.concatenate([emb_user, emb_item], axis=0)
  x1 = _layer(row, col, adj_vals, x0)
  x2 = _layer(row, col, adj_vals, x1)
  x3 = _layer(row, col, adj_vals, x2)
  return _final(x0, x1, x2, x3, users, items)


# speedup vs baseline: 6.1488x; 1.0395x over previous
import functools
import jax
import jax.numpy as jnp
from jax import lax
from jax.experimental import pallas as pl
from jax.experimental.pallas import tpu as pltpu
from jax.experimental.pallas import tpu_sc as plsc

_N_USER = 20000
_N_ITEM = 30000
_N = _N_USER + _N_ITEM
_E = 800000
_D = 64
_B = 4096
_NC, _NS = 2, 16
_RPC = 25600
_SPR = _RPC // _NS
_WCH = 80
_EPS = _E // _NS
_EC = 400
_ET = 80
_NT = _EC // _ET
_NCH = _EPS // _EC
_BPW = _B // (_NC * _NS)

_CAP = _EPS + _EC
_SG = 816

_mesh = plsc.VectorSubcoreMesh(core_axis_name="c", subcore_axis_name="s")


def _compact_body(row_h, col_h, val_h, ccol_h, cval_h, clidx_h, cnt_h,
                  rbuf, cbuf, vbuf, scol, sval, slidx, cntv, esem):
  c = lax.axis_index("c")
  s = lax.axis_index("s")
  base = c * _RPC
  e_base = s * _EPS

  def load_group(g, p):
    e0 = e_base + g * _EC
    pltpu.async_copy(row_h.at[pl.ds(e0, _EC)], rbuf.at[p], esem)
    pltpu.async_copy(col_h.at[pl.ds(e0, _EC)], cbuf.at[p], esem)
    pltpu.async_copy(val_h.at[pl.ds(e0, _EC)], vbuf.at[p], esem)

  def wait_group(p):
    pltpu.make_async_copy(row_h.at[pl.ds(0, _EC)], rbuf.at[p], esem).wait()
    pltpu.make_async_copy(col_h.at[pl.ds(0, _EC)], cbuf.at[p], esem).wait()
    pltpu.make_async_copy(val_h.at[pl.ds(0, _EC)], vbuf.at[p], esem).wait()

  load_group(0, 0)
  load_group(1, 1)

  def grp(g, carry):
    w, off = carry
    gp = lax.rem(g, 2)
    wait_group(gp)
    for j in range(_EC // 16):
      r = rbuf[gp, pl.ds(j * 16, 16)]
      li = r - base
      owned = (li >= 0) & (li < _RPC)
      plsc.store_compressed(scol.at[pl.ds(w, 16)],
                            cbuf[gp, pl.ds(j * 16, 16)], mask=owned)
      plsc.store_compressed(sval.at[pl.ds(w, 16)],
                            vbuf[gp, pl.ds(j * 16, 16)], mask=owned)
      plsc.store_compressed(slidx.at[pl.ds(w, 16)], li, mask=owned)
      w = w + plsc.all_reduce_population_count(owned)[0]

    @pl.when(g + 2 < _NCH)
    def _prefetch():
      load_group(g + 2, gp)

    def flush(wo):
      w_, off_ = wo
      off_ = pl.multiple_of(off_, 8)
      pltpu.sync_copy(scol.at[pl.ds(0, _EC)],
                      ccol_h.at[c, s, pl.ds(off_, _EC)])
      pltpu.sync_copy(sval.at[pl.ds(0, _EC)],
                      cval_h.at[c, s, pl.ds(off_, _EC)])
      pltpu.sync_copy(slidx.at[pl.ds(0, _EC)],
                      clidx_h.at[c, s, pl.ds(off_, _EC)])
      for j in range(_EC // 16):
        sl_src = pl.ds(_EC + j * 16, 16)
        sl_dst = pl.ds(j * 16, 16)
        scol[sl_dst] = scol[sl_src]
        sval[sl_dst] = sval[sl_src]
        slidx[sl_dst] = slidx[sl_src]
      return (w_ - _EC, off_ + _EC)

    return lax.cond(w >= _EC, flush, lambda wo: wo, (w, off))

  w, off = lax.fori_loop(0, _NCH, grp, (0, 0))

  lane = lax.iota(jnp.int32, 16)
  for j in range(_EC // 16):
    sl = pl.ds(j * 16, 16)
    m = (j * 16 + lane) < w
    scol[sl] = jnp.where(m, scol[sl], 0)
    sval[sl] = jnp.where(m, sval[sl], jnp.float32(0.0))
    slidx[sl] = jnp.where(m, slidx[sl], 0)
  off = pl.multiple_of(off, 8)
  pltpu.sync_copy(scol.at[pl.ds(0, _EC)], ccol_h.at[c, s, pl.ds(off, _EC)])
  pltpu.sync_copy(sval.at[pl.ds(0, _EC)], cval_h.at[c, s, pl.ds(off, _EC)])
  pltpu.sync_copy(slidx.at[pl.ds(0, _EC)], clidx_h.at[c, s, pl.ds(off, _EC)])
  cntv[pl.ds(0, 16)] = jnp.full((16,), off + _EC)
  pltpu.sync_copy(cntv, cnt_h.at[c, s])


_compact = functools.partial(
    pl.kernel,
    out_type=(
        jax.ShapeDtypeStruct((_NC, _NS, _CAP), jnp.int32),
        jax.ShapeDtypeStruct((_NC, _NS, _CAP), jnp.float32),
        jax.ShapeDtypeStruct((_NC, _NS, _CAP), jnp.int32),
        jax.ShapeDtypeStruct((_NC, _NS, 16), jnp.int32),
    ),
    scratch_types=[
        pltpu.VMEM((2, _EC), jnp.int32),
        pltpu.VMEM((2, _EC), jnp.int32),
        pltpu.VMEM((2, _EC), jnp.float32),
        pltpu.VMEM((_SG,), jnp.int32),
        pltpu.VMEM((_SG,), jnp.float32),
        pltpu.VMEM((_SG,), jnp.int32),
        pltpu.VMEM((16,), jnp.int32),
        pltpu.SemaphoreType.DMA,
    ],
    mesh=_mesh,
    compiler_params=pltpu.CompilerParams(use_tc_tiling_on_sc=False,
                                         needs_layout_passes=False),
)(_compact_body)


def _layer_body(ccol_h, cval_h, clidx_h, cnt_h, xsrc_h, xdst_h,
                acc, cbuf, vbuf, lidx, rows, cntv, gsem, esem, ssem):
  c = lax.axis_index("c")
  s = lax.axis_index("s")
  pltpu.sync_copy(cnt_h.at[c, s], cntv)
  ng = cntv[pl.ds(0, 16)][0] // _EC
  zero16 = jnp.zeros((16,), jnp.float32)

  def zrow(i, carry):
    for k in range(_D // 16):
      rows[0, i, pl.ds(k * 16, 16)] = zero16
    return carry
  lax.fori_loop(0, _WCH, zrow, 0)

  def zacc(i, carry):
    pltpu.async_copy(rows.at[0],
                     acc.at[pl.ds(s * _SPR + i * _WCH, _WCH)], esem)
    return carry
  lax.fori_loop(0, _SPR // _WCH, zacc, 0)

  def zwait(i, carry):
    pltpu.make_async_copy(rows.at[0],
                          acc.at[pl.ds(s * _SPR, _WCH)], esem).wait()
    return carry
  lax.fori_loop(0, _SPR // _WCH, zwait, 0)

  plsc.subcore_barrier()

  def load_group(g, p):
    e0 = pl.multiple_of(g * _EC, 8)
    pltpu.async_copy(ccol_h.at[c, s, pl.ds(e0, _EC)], cbuf.at[p], esem)
    pltpu.async_copy(cval_h.at[c, s, pl.ds(e0, _EC)], vbuf.at[p], esem)
    for t in range(_NT):
      pltpu.async_copy(clidx_h.at[c, s, pl.ds(e0 + t * _ET, _ET)],
                       lidx.at[p, t], esem)

  def wait_group(p):
    pltpu.make_async_copy(ccol_h.at[c, s, pl.ds(0, _EC)],
                          cbuf.at[p], esem).wait()
    pltpu.make_async_copy(cval_h.at[c, s, pl.ds(0, _EC)],
                          vbuf.at[p], esem).wait()
    for t in range(_NT):
      pltpu.make_async_copy(clidx_h.at[c, s, pl.ds(0, _ET)],
                            lidx.at[p, t], esem).wait()

  def fire_gather(p, t, b):
    pltpu.async_copy(xsrc_h.at[cbuf.at[p, pl.ds(t * _ET, _ET)]],
                     rows.at[b], gsem)

  def wait_gather(b):
    pltpu.make_async_copy(xsrc_h.at[pl.ds(0, _ET)], rows.at[b], gsem).wait()

  def wait_scatter():
    pltpu.make_async_copy(rows.at[0], acc.at[lidx.at[0, 0]], ssem).wait()

  load_group(0, 0)
  wait_group(0)

  @pl.when(ng > 1)
  def _pre():
    load_group(1, 1)
  fire_gather(0, 0, 0)

  def group_body(g, carry):
    gp = lax.rem(g, 3)
    for t in range(_NT):
      m = g * _NT + t
      b = lax.rem(m, 3)
      wait_gather(b)
      @pl.when(m >= 2)
      def _drain():
        wait_scatter()
      if t < _NT - 1:
        fire_gather(gp, t + 1, lax.rem(m + 1, 3))
      else:
        gn = g + 1
        gnp = lax.rem(gn, 3)

        @pl.when(gn < ng)
        def _boundary():
          wait_group(gnp)

          @pl.when(gn + 1 < ng)
          def _prefetch():
            load_group(gn + 1, lax.rem(gn + 1, 3))
          fire_gather(gnp, 0, lax.rem(m + 1, 3))
      def scale(j, carry2):
        vv = vbuf[gp, pl.ds(t * _ET + j * 16, 16)]
        for l in range(16):
          e = j * 16 + l
          vsp = jnp.full((16,), vv[l])
          for k in range(_D // 16):
            rows[b, e, pl.ds(k * 16, 16)] = (
                rows[b, e, pl.ds(k * 16, 16)] * vsp)
        return carry2
      lax.fori_loop(0, _ET // 16, scale, 0)
      pltpu.async_copy(rows.at[b], acc.at[lidx.at[gp, t]], ssem, add=True)
    return carry
  lax.fori_loop(0, ng, group_body, 0)
  wait_scatter()
  wait_scatter()

  plsc.subcore_barrier()

  g0 = c * _RPC + s * _SPR
  nch = jnp.minimum(_SPR, jnp.maximum(0, _N - g0)) // _WCH

  def wback(i, carry):
    pltpu.async_copy(acc.at[pl.ds(s * _SPR + i * _WCH, _WCH)],
                     xdst_h.at[pl.ds(g0 + i * _WCH, _WCH)], esem)
    return carry
  lax.fori_loop(0, nch, wback, 0)

  def wbwait(i, carry):
    pltpu.make_async_copy(acc.at[pl.ds(s * _SPR, _WCH)],
                          xdst_h.at[pl.ds(g0, _WCH)], esem).wait()
    return carry
  lax.fori_loop(0, nch, wbwait, 0)


_layer = functools.partial(
    pl.kernel,
    out_type=jax.ShapeDtypeStruct((_N, _D), jnp.float32),
    scratch_types=[
        pltpu.VMEM_SHARED((_RPC, _D), jnp.float32),
        pltpu.VMEM((3, _EC), jnp.int32),
        pltpu.VMEM((3, _EC), jnp.float32),
        pltpu.VMEM((3, _NT, _ET), jnp.int32),
        pltpu.VMEM((3, _ET, _D), jnp.float32),
        pltpu.VMEM((16,), jnp.int32),
        pltpu.SemaphoreType.DMA,
        pltpu.SemaphoreType.DMA,
        pltpu.SemaphoreType.DMA,
    ],
    mesh=_mesh,
    compiler_params=pltpu.CompilerParams(use_tc_tiling_on_sc=False, needs_layout_passes=False),
)(_layer_body)


def _final_body(x0_h, x1_h, x2_h, x3_h, users_h, items_h, gamma_h,
                ubuf, ibuf, usum, isum, tbuf, gbuf, sem):
  c = lax.axis_index("c")
  s = lax.axis_index("s")
  wid = s * _NC + c
  b0 = wid * _BPW
  pltpu.sync_copy(users_h.at[pl.ds(b0, _BPW)], ubuf)
  pltpu.sync_copy(items_h.at[pl.ds(b0, _BPW)], ibuf)
  for j in range(_BPW // 16):
    ibuf[pl.ds(j * 16, 16)] = ibuf[pl.ds(j * 16, 16)] + _N_USER

  pltpu.async_copy(x0_h.at[ubuf], usum, sem).wait()
  pltpu.async_copy(x0_h.at[ibuf], isum, sem).wait()
  for xl_h in (x1_h, x2_h, x3_h):
    pltpu.async_copy(xl_h.at[ubuf], tbuf, sem).wait()

    def accu(i, carry):
      for k in range(_D // 16):
        sl = pl.ds(k * 16, 16)
        usum[i, sl] = usum[i, sl] + tbuf[i, sl]
      return carry
    lax.fori_loop(0, _BPW, accu, 0)

    pltpu.async_copy(xl_h.at[ibuf], tbuf, sem).wait()

    def acci(i, carry):
      for k in range(_D // 16):
        sl = pl.ds(k * 16, 16)
        isum[i, sl] = isum[i, sl] + tbuf[i, sl]
      return carry
    lax.fori_loop(0, _BPW, acci, 0)

  lane = lax.iota(jnp.int32, 16)

  def group(g, carry):
    out16 = jnp.zeros((16,), jnp.float32)
    for l in range(16):
      i = g * 16 + l
      acc2 = jnp.zeros((16,), jnp.float32)
      for k in range(_D // 16):
        sl = pl.ds(k * 16, 16)
        acc2 = acc2 + usum[i, sl] * isum[i, sl]
      dot = jnp.sum(acc2) * jnp.float32(0.0625)
      out16 = jnp.where(lane == l, jnp.full((16,), dot), out16)
    gbuf[pl.ds(g * 16, 16)] = out16
    return carry
  lax.fori_loop(0, _BPW // 16, group, 0)

  pltpu.sync_copy(gbuf, gamma_h.at[pl.ds(b0, _BPW)])


_final = functools.partial(
    pl.kernel,
    out_type=jax.ShapeDtypeStruct((_B,), jnp.float32),
    scratch_types=[
        pltpu.VMEM((_BPW,), jnp.int32),
        pltpu.VMEM((_BPW,), jnp.int32),
        pltpu.VMEM((_BPW, _D), jnp.float32),
        pltpu.VMEM((_BPW, _D), jnp.float32),
        pltpu.VMEM((_BPW, _D), jnp.float32),
        pltpu.VMEM((_BPW,), jnp.float32),
        pltpu.SemaphoreType.DMA,
    ],
    mesh=_mesh,
    compiler_params=pltpu.CompilerParams(use_tc_tiling_on_sc=False, needs_layout_passes=False),
)(_final_body)


def kernel(edge_index, adj_vals, users, items, emb_user, emb_item):
  row = edge_index[0]
  col = edge_index[1]
  x0 = jnp.concatenate([emb_user, emb_item], axis=0)
  ccol, cval, clidx, cnt = _compact(row, col, adj_vals)
  x1 = _layer(ccol, cval, clidx, cnt, x0)
  x2 = _layer(ccol, cval, clidx, cnt, x1)
  x3 = _layer(ccol, cval, clidx, cnt, x2)
  return _final(x0, x1, x2, x3, users, items)


# speedup vs baseline: 6.1561x; 1.0012x over previous
import functools
import jax
import jax.numpy as jnp
from jax import lax
from jax.experimental import pallas as pl
from jax.experimental.pallas import tpu as pltpu
from jax.experimental.pallas import tpu_sc as plsc

_N_USER = 20000
_N_ITEM = 30000
_N = _N_USER + _N_ITEM
_E = 800000
_D = 64
_B = 4096
_NC, _NS = 2, 16
_RPC = 25600
_SPR = _RPC // _NS
_WCH = 80
_EPS = _E // _NS
_EC = 400
_ET = 80
_NT = _EC // _ET
_NCH = _EPS // _EC
_BPW = _B // (_NC * _NS)

_CAP = _EPS + _EC
_SG = 816

_mesh = plsc.VectorSubcoreMesh(core_axis_name="c", subcore_axis_name="s")


def _compact_body(row_h, col_h, val_h, ccol_h, cval_h, clidx_h, cnt_h,
                  rbuf, cbuf, vbuf, scol, sval, slidx, cntv, esem):
  c = lax.axis_index("c")
  s = lax.axis_index("s")
  base = c * _RPC
  e_base = s * _EPS

  def load_group(g, p):
    e0 = e_base + g * _EC
    pltpu.async_copy(row_h.at[pl.ds(e0, _EC)], rbuf.at[p], esem)
    pltpu.async_copy(col_h.at[pl.ds(e0, _EC)], cbuf.at[p], esem)
    pltpu.async_copy(val_h.at[pl.ds(e0, _EC)], vbuf.at[p], esem)

  def wait_group(p):
    pltpu.make_async_copy(row_h.at[pl.ds(0, _EC)], rbuf.at[p], esem).wait()
    pltpu.make_async_copy(col_h.at[pl.ds(0, _EC)], cbuf.at[p], esem).wait()
    pltpu.make_async_copy(val_h.at[pl.ds(0, _EC)], vbuf.at[p], esem).wait()

  load_group(0, 0)
  load_group(1, 1)

  def grp(g, carry):
    w, off = carry
    gp = lax.rem(g, 2)
    wait_group(gp)
    for j in range(_EC // 16):
      r = rbuf[gp, pl.ds(j * 16, 16)]
      li = r - base
      owned = (li >= 0) & (li < _RPC)
      plsc.store_compressed(scol.at[pl.ds(w, 16)],
                            cbuf[gp, pl.ds(j * 16, 16)], mask=owned)
      plsc.store_compressed(sval.at[pl.ds(w, 16)],
                            vbuf[gp, pl.ds(j * 16, 16)], mask=owned)
      plsc.store_compressed(slidx.at[pl.ds(w, 16)], li, mask=owned)
      w = w + plsc.all_reduce_population_count(owned)[0]

    @pl.when(g + 2 < _NCH)
    def _prefetch():
      load_group(g + 2, gp)

    def flush(wo):
      w_, off_ = wo
      off_ = pl.multiple_of(off_, 8)
      pltpu.sync_copy(scol.at[pl.ds(0, _EC)],
                      ccol_h.at[c, s, pl.ds(off_, _EC)])
      pltpu.sync_copy(sval.at[pl.ds(0, _EC)],
                      cval_h.at[c, s, pl.ds(off_, _EC)])
      pltpu.sync_copy(slidx.at[pl.ds(0, _EC)],
                      clidx_h.at[c, s, pl.ds(off_, _EC)])
      for j in range(_EC // 16):
        sl_src = pl.ds(_EC + j * 16, 16)
        sl_dst = pl.ds(j * 16, 16)
        scol[sl_dst] = scol[sl_src]
        sval[sl_dst] = sval[sl_src]
        slidx[sl_dst] = slidx[sl_src]
      return (w_ - _EC, off_ + _EC)

    return lax.cond(w >= _EC, flush, lambda wo: wo, (w, off))

  w, off = lax.fori_loop(0, _NCH, grp, (0, 0))

  lane = lax.iota(jnp.int32, 16)
  for j in range(_EC // 16):
    sl = pl.ds(j * 16, 16)
    m = (j * 16 + lane) < w
    scol[sl] = jnp.where(m, scol[sl], 0)
    sval[sl] = jnp.where(m, sval[sl], jnp.float32(0.0))
    slidx[sl] = jnp.where(m, slidx[sl], 0)
  off = pl.multiple_of(off, 8)
  pltpu.sync_copy(scol.at[pl.ds(0, _EC)], ccol_h.at[c, s, pl.ds(off, _EC)])
  pltpu.sync_copy(sval.at[pl.ds(0, _EC)], cval_h.at[c, s, pl.ds(off, _EC)])
  pltpu.sync_copy(slidx.at[pl.ds(0, _EC)], clidx_h.at[c, s, pl.ds(off, _EC)])
  cntv[pl.ds(0, 16)] = jnp.full((16,), off + _EC)
  pltpu.sync_copy(cntv, cnt_h.at[c, s])


_compact = functools.partial(
    pl.kernel,
    out_type=(
        jax.ShapeDtypeStruct((_NC, _NS, _CAP), jnp.int32),
        jax.ShapeDtypeStruct((_NC, _NS, _CAP), jnp.float32),
        jax.ShapeDtypeStruct((_NC, _NS, _CAP), jnp.int32),
        jax.ShapeDtypeStruct((_NC, _NS, 16), jnp.int32),
    ),
    scratch_types=[
        pltpu.VMEM((2, _EC), jnp.int32),
        pltpu.VMEM((2, _EC), jnp.int32),
        pltpu.VMEM((2, _EC), jnp.float32),
        pltpu.VMEM((_SG,), jnp.int32),
        pltpu.VMEM((_SG,), jnp.float32),
        pltpu.VMEM((_SG,), jnp.int32),
        pltpu.VMEM((16,), jnp.int32),
        pltpu.SemaphoreType.DMA,
    ],
    mesh=_mesh,
    compiler_params=pltpu.CompilerParams(use_tc_tiling_on_sc=False,
                                         needs_layout_passes=False),
)(_compact_body)


def _layer_body(ccol_h, cval_h, clidx_h, cnt_h, xsrc_h, xdst_h,
                acc, cbuf, vbuf, lidx, rows, cntv, gsem, esem, ssem):
  c = lax.axis_index("c")
  s = lax.axis_index("s")
  pltpu.sync_copy(cnt_h.at[c, s], cntv)
  ng = cntv[pl.ds(0, 16)][0] // _EC
  zero16 = jnp.zeros((16,), jnp.float32)

  def zrow(i, carry):
    for k in range(_D // 16):
      rows[0, i, pl.ds(k * 16, 16)] = zero16
    return carry
  lax.fori_loop(0, _WCH, zrow, 0)

  def zacc(i, carry):
    pltpu.async_copy(rows.at[0],
                     acc.at[pl.ds(s * _SPR + i * _WCH, _WCH)], esem)
    return carry
  lax.fori_loop(0, _SPR // _WCH, zacc, 0)

  def zwait(i, carry):
    pltpu.make_async_copy(rows.at[0],
                          acc.at[pl.ds(s * _SPR, _WCH)], esem).wait()
    return carry
  lax.fori_loop(0, _SPR // _WCH, zwait, 0)

  plsc.subcore_barrier()

  def load_group(g, p):
    e0 = pl.multiple_of(g * _EC, 8)
    pltpu.async_copy(ccol_h.at[c, s, pl.ds(e0, _EC)], cbuf.at[p], esem)
    pltpu.async_copy(cval_h.at[c, s, pl.ds(e0, _EC)], vbuf.at[p], esem)
    for t in range(_NT):
      pltpu.async_copy(clidx_h.at[c, s, pl.ds(e0 + t * _ET, _ET)],
                       lidx.at[p, t], esem)

  def wait_group(p):
    pltpu.make_async_copy(ccol_h.at[c, s, pl.ds(0, _EC)],
                          cbuf.at[p], esem).wait()
    pltpu.make_async_copy(cval_h.at[c, s, pl.ds(0, _EC)],
                          vbuf.at[p], esem).wait()
    for t in range(_NT):
      pltpu.make_async_copy(clidx_h.at[c, s, pl.ds(0, _ET)],
                            lidx.at[p, t], esem).wait()

  def fire_gather(p, t, b):
    pltpu.async_copy(xsrc_h.at[cbuf.at[p, pl.ds(t * _ET, _ET)]],
                     rows.at[b], gsem)

  def wait_gather(b):
    pltpu.make_async_copy(xsrc_h.at[pl.ds(0, _ET)], rows.at[b], gsem).wait()

  def wait_scatter():
    pltpu.make_async_copy(rows.at[0], acc.at[lidx.at[0, 0]], ssem).wait()

  load_group(0, 0)
  wait_group(0)

  @pl.when(ng > 1)
  def _pre():
    load_group(1, 1)
  fire_gather(0, 0, 0)

  def group_body(g, carry):
    gp = lax.rem(g, 3)
    for t in range(_NT):
      m = g * _NT + t
      b = lax.rem(m, 3)
      wait_gather(b)
      @pl.when(m >= 2)
      def _drain():
        wait_scatter()
      if t < _NT - 1:
        fire_gather(gp, t + 1, lax.rem(m + 1, 3))
      else:
        gn = g + 1
        gnp = lax.rem(gn, 3)

        @pl.when(gn < ng)
        def _boundary():
          wait_group(gnp)

          @pl.when(gn + 1 < ng)
          def _prefetch():
            load_group(gn + 1, lax.rem(gn + 1, 3))
          fire_gather(gnp, 0, lax.rem(m + 1, 3))
      def scale(j, carry2):
        vv = vbuf[gp, pl.ds(t * _ET + j * 16, 16)]
        for l in range(16):
          e = j * 16 + l
          vsp = lax.gather(
              vv, jnp.full((16, 1), l, jnp.int32),
              lax.GatherDimensionNumbers(offset_dims=(),
                                         collapsed_slice_dims=(0,),
                                         start_index_map=(0,)),
              (1,), mode=lax.GatherScatterMode.PROMISE_IN_BOUNDS)
          for k in range(_D // 16):
            rows[b, e, pl.ds(k * 16, 16)] = (
                rows[b, e, pl.ds(k * 16, 16)] * vsp)
        return carry2
      lax.fori_loop(0, _ET // 16, scale, 0)
      pltpu.async_copy(rows.at[b], acc.at[lidx.at[gp, t]], ssem, add=True)
    return carry
  lax.fori_loop(0, ng, group_body, 0)
  wait_scatter()
  wait_scatter()

  plsc.subcore_barrier()

  g0 = c * _RPC + s * _SPR
  nch = jnp.minimum(_SPR, jnp.maximum(0, _N - g0)) // _WCH

  def wback(i, carry):
    pltpu.async_copy(acc.at[pl.ds(s * _SPR + i * _WCH, _WCH)],
                     xdst_h.at[pl.ds(g0 + i * _WCH, _WCH)], esem)
    return carry
  lax.fori_loop(0, nch, wback, 0)

  def wbwait(i, carry):
    pltpu.make_async_copy(acc.at[pl.ds(s * _SPR, _WCH)],
                          xdst_h.at[pl.ds(g0, _WCH)], esem).wait()
    return carry
  lax.fori_loop(0, nch, wbwait, 0)


_layer = functools.partial(
    pl.kernel,
    out_type=jax.ShapeDtypeStruct((_N, _D), jnp.float32),
    scratch_types=[
        pltpu.VMEM_SHARED((_RPC, _D), jnp.float32),
        pltpu.VMEM((3, _EC), jnp.int32),
        pltpu.VMEM((3, _EC), jnp.float32),
        pltpu.VMEM((3, _NT, _ET), jnp.int32),
        pltpu.VMEM((3, _ET, _D), jnp.float32),
        pltpu.VMEM((16,), jnp.int32),
        pltpu.SemaphoreType.DMA,
        pltpu.SemaphoreType.DMA,
        pltpu.SemaphoreType.DMA,
    ],
    mesh=_mesh,
    compiler_params=pltpu.CompilerParams(use_tc_tiling_on_sc=False, needs_layout_passes=False),
)(_layer_body)


def _final_body(x0_h, x1_h, x2_h, x3_h, users_h, items_h, gamma_h,
                ubuf, ibuf, usum, isum, tbuf, gbuf, sem):
  c = lax.axis_index("c")
  s = lax.axis_index("s")
  wid = s * _NC + c
  b0 = wid * _BPW
  pltpu.sync_copy(users_h.at[pl.ds(b0, _BPW)], ubuf)
  pltpu.sync_copy(items_h.at[pl.ds(b0, _BPW)], ibuf)
  for j in range(_BPW // 16):
    ibuf[pl.ds(j * 16, 16)] = ibuf[pl.ds(j * 16, 16)] + _N_USER

  pltpu.async_copy(x0_h.at[ubuf], usum, sem).wait()
  pltpu.async_copy(x0_h.at[ibuf], isum, sem).wait()
  for xl_h in (x1_h, x2_h, x3_h):
    pltpu.async_copy(xl_h.at[ubuf], tbuf, sem).wait()

    def accu(i, carry):
      for k in range(_D // 16):
        sl = pl.ds(k * 16, 16)
        usum[i, sl] = usum[i, sl] + tbuf[i, sl]
      return carry
    lax.fori_loop(0, _BPW, accu, 0)

    pltpu.async_copy(xl_h.at[ibuf], tbuf, sem).wait()

    def acci(i, carry):
      for k in range(_D // 16):
        sl = pl.ds(k * 16, 16)
        isum[i, sl] = isum[i, sl] + tbuf[i, sl]
      return carry
    lax.fori_loop(0, _BPW, acci, 0)

  lane = lax.iota(jnp.int32, 16)

  def group(g, carry):
    out16 = jnp.zeros((16,), jnp.float32)
    for l in range(16):
      i = g * 16 + l
      acc2 = jnp.zeros((16,), jnp.float32)
      for k in range(_D // 16):
        sl = pl.ds(k * 16, 16)
        acc2 = acc2 + usum[i, sl] * isum[i, sl]
      dot = jnp.sum(acc2) * jnp.float32(0.0625)
      out16 = jnp.where(lane == l, jnp.full((16,), dot), out16)
    gbuf[pl.ds(g * 16, 16)] = out16
    return carry
  lax.fori_loop(0, _BPW // 16, group, 0)

  pltpu.sync_copy(gbuf, gamma_h.at[pl.ds(b0, _BPW)])


_final = functools.partial(
    pl.kernel,
    out_type=jax.ShapeDtypeStruct((_B,), jnp.float32),
    scratch_types=[
        pltpu.VMEM((_BPW,), jnp.int32),
        pltpu.VMEM((_BPW,), jnp.int32),
        pltpu.VMEM((_BPW, _D), jnp.float32),
        pltpu.VMEM((_BPW, _D), jnp.float32),
        pltpu.VMEM((_BPW, _D), jnp.float32),
        pltpu.VMEM((_BPW,), jnp.float32),
        pltpu.SemaphoreType.DMA,
    ],
    mesh=_mesh,
    compiler_params=pltpu.CompilerParams(use_tc_tiling_on_sc=False, needs_layout_passes=False),
)(_final_body)


def kernel(edge_index, adj_vals, users, items, emb_user, emb_item):
  row = edge_index[0]
  col = edge_index[1]
  x0 = jnp.concatenate([emb_user, emb_item], axis=0)
  ccol, cval, clidx, cnt = _compact(row, col, adj_vals)
  x1 = _layer(ccol, cval, clidx, cnt, x0)
  x2 = _layer(ccol, cval, clidx, cnt, x1)
  x3 = _layer(ccol, cval, clidx, cnt, x2)
  return _final(x0, x1, x2, x3, users, items)


# speedup vs baseline: 6.6528x; 1.0807x over previous
import functools
import jax
import jax.numpy as jnp
from jax import lax
from jax.experimental import pallas as pl
from jax.experimental.pallas import tpu as pltpu
from jax.experimental.pallas import tpu_sc as plsc

_N_USER = 20000
_N_ITEM = 30000
_N = _N_USER + _N_ITEM
_E = 800000
_D = 64
_B = 4096
_NC, _NS = 2, 16
_RPC = 25600
_SPR = _RPC // _NS
_WCH = 80
_EPS = _E // _NS
_EC = 400
_ET = 80
_NT = _EC // _ET
_NCH = _EPS // _EC
_BPW = _B // (_NC * _NS)

_CAP = _EPS + _EC
_SG = 816

_mesh = plsc.VectorSubcoreMesh(core_axis_name="c", subcore_axis_name="s")


def _compact_body(row_h, col_h, val_h, ccol_h, cval_h, clidx_h, cnt_h,
                  rbuf, cbuf, vbuf, scol, sval, slidx, cntv, esem):
  c = lax.axis_index("c")
  s = lax.axis_index("s")
  base = c * _RPC
  e_base = s * _EPS

  def load_group(g, p):
    e0 = e_base + g * _EC
    pltpu.async_copy(row_h.at[pl.ds(e0, _EC)], rbuf.at[p], esem)
    pltpu.async_copy(col_h.at[pl.ds(e0, _EC)], cbuf.at[p], esem)
    pltpu.async_copy(val_h.at[pl.ds(e0, _EC)], vbuf.at[p], esem)

  def wait_group(p):
    pltpu.make_async_copy(row_h.at[pl.ds(0, _EC)], rbuf.at[p], esem).wait()
    pltpu.make_async_copy(col_h.at[pl.ds(0, _EC)], cbuf.at[p], esem).wait()
    pltpu.make_async_copy(val_h.at[pl.ds(0, _EC)], vbuf.at[p], esem).wait()

  load_group(0, 0)
  load_group(1, 1)

  def grp(g, carry):
    w, off = carry
    gp = lax.rem(g, 2)
    wait_group(gp)
    for j in range(_EC // 16):
      r = rbuf[gp, pl.ds(j * 16, 16)]
      li = r - base
      owned = (li >= 0) & (li < _RPC)
      plsc.store_compressed(scol.at[pl.ds(w, 16)],
                            cbuf[gp, pl.ds(j * 16, 16)], mask=owned)
      plsc.store_compressed(sval.at[pl.ds(w, 16)],
                            vbuf[gp, pl.ds(j * 16, 16)], mask=owned)
      plsc.store_compressed(slidx.at[pl.ds(w, 16)], li, mask=owned)
      w = w + plsc.all_reduce_population_count(owned)[0]

    @pl.when(g + 2 < _NCH)
    def _prefetch():
      load_group(g + 2, gp)

    def flush(wo):
      w_, off_ = wo
      off_ = pl.multiple_of(off_, 8)
      pltpu.sync_copy(scol.at[pl.ds(0, _EC)],
                      ccol_h.at[c, s, pl.ds(off_, _EC)])
      pltpu.sync_copy(sval.at[pl.ds(0, _EC)],
                      cval_h.at[c, s, pl.ds(off_, _EC)])
      pltpu.sync_copy(slidx.at[pl.ds(0, _EC)],
                      clidx_h.at[c, s, pl.ds(off_, _EC)])
      for j in range(_EC // 16):
        sl_src = pl.ds(_EC + j * 16, 16)
        sl_dst = pl.ds(j * 16, 16)
        scol[sl_dst] = scol[sl_src]
        sval[sl_dst] = sval[sl_src]
        slidx[sl_dst] = slidx[sl_src]
      return (w_ - _EC, off_ + _EC)

    return lax.cond(w >= _EC, flush, lambda wo: wo, (w, off))

  w, off = lax.fori_loop(0, _NCH, grp, (0, 0))

  lane = lax.iota(jnp.int32, 16)
  for j in range(_EC // 16):
    sl = pl.ds(j * 16, 16)
    m = (j * 16 + lane) < w
    scol[sl] = jnp.where(m, scol[sl], 0)
    sval[sl] = jnp.where(m, sval[sl], jnp.float32(0.0))
    slidx[sl] = jnp.where(m, slidx[sl], 0)
  off = pl.multiple_of(off, 8)
  pltpu.sync_copy(scol.at[pl.ds(0, _EC)], ccol_h.at[c, s, pl.ds(off, _EC)])
  pltpu.sync_copy(sval.at[pl.ds(0, _EC)], cval_h.at[c, s, pl.ds(off, _EC)])
  pltpu.sync_copy(slidx.at[pl.ds(0, _EC)], clidx_h.at[c, s, pl.ds(off, _EC)])
  cntv[pl.ds(0, 16)] = jnp.full((16,), off + _EC)
  pltpu.sync_copy(cntv, cnt_h.at[c, s])


_compact = functools.partial(
    pl.kernel,
    out_type=(
        jax.ShapeDtypeStruct((_NC, _NS, _CAP), jnp.int32),
        jax.ShapeDtypeStruct((_NC, _NS, _CAP), jnp.float32),
        jax.ShapeDtypeStruct((_NC, _NS, _CAP), jnp.int32),
        jax.ShapeDtypeStruct((_NC, _NS, 16), jnp.int32),
    ),
    scratch_types=[
        pltpu.VMEM((2, _EC), jnp.int32),
        pltpu.VMEM((2, _EC), jnp.int32),
        pltpu.VMEM((2, _EC), jnp.float32),
        pltpu.VMEM((_SG,), jnp.int32),
        pltpu.VMEM((_SG,), jnp.float32),
        pltpu.VMEM((_SG,), jnp.int32),
        pltpu.VMEM((16,), jnp.int32),
        pltpu.SemaphoreType.DMA,
    ],
    mesh=_mesh,
    compiler_params=pltpu.CompilerParams(use_tc_tiling_on_sc=False,
                                         needs_layout_passes=False),
)(_compact_body)


def _layer_body(ccol_h, cval_h, clidx_h, cnt_h, xsrc_h, xdst_h,
                acc, cbuf, vbuf, lidx, rows, cntv, gsem, esem, ssem):
  c = lax.axis_index("c")
  s = lax.axis_index("s")
  pltpu.sync_copy(cnt_h.at[c, s], cntv)
  ng = cntv[pl.ds(0, 16)][0] // _EC
  zero16 = jnp.zeros((16,), jnp.float32)

  def zrow(i, carry):
    for k in range(_D // 16):
      rows[0, i, pl.ds(k * 16, 16)] = zero16
    return carry
  lax.fori_loop(0, _WCH, zrow, 0)

  def zacc(i, carry):
    pltpu.async_copy(rows.at[0],
                     acc.at[pl.ds(s * _SPR + i * _WCH, _WCH)], esem)
    return carry
  lax.fori_loop(0, _SPR // _WCH, zacc, 0)

  def zwait(i, carry):
    pltpu.make_async_copy(rows.at[0],
                          acc.at[pl.ds(s * _SPR, _WCH)], esem).wait()
    return carry
  lax.fori_loop(0, _SPR // _WCH, zwait, 0)

  plsc.subcore_barrier()

  def load_group(g, p):
    e0 = pl.multiple_of(g * _EC, 8)
    pltpu.async_copy(ccol_h.at[c, s, pl.ds(e0, _EC)], cbuf.at[p], esem)
    pltpu.async_copy(cval_h.at[c, s, pl.ds(e0, _EC)], vbuf.at[p], esem)
    for t in range(_NT):
      pltpu.async_copy(clidx_h.at[c, s, pl.ds(e0 + t * _ET, _ET)],
                       lidx.at[p, t], esem)

  def wait_group(p):
    pltpu.make_async_copy(ccol_h.at[c, s, pl.ds(0, _EC)],
                          cbuf.at[p], esem).wait()
    pltpu.make_async_copy(cval_h.at[c, s, pl.ds(0, _EC)],
                          vbuf.at[p], esem).wait()
    for t in range(_NT):
      pltpu.make_async_copy(clidx_h.at[c, s, pl.ds(0, _ET)],
                            lidx.at[p, t], esem).wait()

  def fire_gather(p, t, b):
    pltpu.async_copy(xsrc_h.at[cbuf.at[p, pl.ds(t * _ET, _ET)]],
                     rows.at[b], gsem)

  def wait_gather(b):
    pltpu.make_async_copy(xsrc_h.at[pl.ds(0, _ET)], rows.at[b], gsem).wait()

  def wait_scatter():
    pltpu.make_async_copy(rows.at[0], acc.at[lidx.at[0, 0]], ssem).wait()

  load_group(0, 0)
  wait_group(0)

  @pl.when(ng > 1)
  def _pre():
    load_group(1, 1)
  fire_gather(0, 0, 0)
  fire_gather(0, 1, 1)

  def group_body(g, carry):
    gp = lax.rem(g, 3)
    for t in range(_NT):
      m = g * _NT + t
      b = lax.rem(m, 4)
      wait_gather(b)
      @pl.when(m >= 2)
      def _drain():
        wait_scatter()
      if t < _NT - 2:
        fire_gather(gp, t + 2, lax.rem(m + 2, 4))
      elif t == _NT - 2:
        gn = g + 1
        gnp = lax.rem(gn, 3)

        @pl.when(gn < ng)
        def _boundary():
          wait_group(gnp)

          @pl.when(gn + 1 < ng)
          def _prefetch():
            load_group(gn + 1, lax.rem(gn + 1, 3))
          fire_gather(gnp, 0, lax.rem(m + 2, 4))
      else:
        gn = g + 1
        gnp = lax.rem(gn, 3)

        @pl.when(gn < ng)
        def _next2():
          fire_gather(gnp, 1, lax.rem(m + 2, 4))
      def scale(j, carry2):
        vv = vbuf[gp, pl.ds(t * _ET + j * 16, 16)]
        for l in range(16):
          e = j * 16 + l
          vsp = lax.gather(
              vv, jnp.full((16, 1), l, jnp.int32),
              lax.GatherDimensionNumbers(offset_dims=(),
                                         collapsed_slice_dims=(0,),
                                         start_index_map=(0,)),
              (1,), mode=lax.GatherScatterMode.PROMISE_IN_BOUNDS)
          for k in range(_D // 16):
            rows[b, e, pl.ds(k * 16, 16)] = (
                rows[b, e, pl.ds(k * 16, 16)] * vsp)
        return carry2
      lax.fori_loop(0, _ET // 16, scale, 0)
      pltpu.async_copy(rows.at[b], acc.at[lidx.at[gp, t]], ssem, add=True)
    return carry
  lax.fori_loop(0, ng, group_body, 0)
  wait_scatter()
  wait_scatter()

  plsc.subcore_barrier()

  g0 = c * _RPC + s * _SPR
  nch = jnp.minimum(_SPR, jnp.maximum(0, _N - g0)) // _WCH

  def wback(i, carry):
    pltpu.async_copy(acc.at[pl.ds(s * _SPR + i * _WCH, _WCH)],
                     xdst_h.at[pl.ds(g0 + i * _WCH, _WCH)], esem)
    return carry
  lax.fori_loop(0, nch, wback, 0)

  def wbwait(i, carry):
    pltpu.make_async_copy(acc.at[pl.ds(s * _SPR, _WCH)],
                          xdst_h.at[pl.ds(g0, _WCH)], esem).wait()
    return carry
  lax.fori_loop(0, nch, wbwait, 0)


_layer = functools.partial(
    pl.kernel,
    out_type=jax.ShapeDtypeStruct((_N, _D), jnp.float32),
    scratch_types=[
        pltpu.VMEM_SHARED((_RPC, _D), jnp.float32),
        pltpu.VMEM((3, _EC), jnp.int32),
        pltpu.VMEM((3, _EC), jnp.float32),
        pltpu.VMEM((3, _NT, _ET), jnp.int32),
        pltpu.VMEM((4, _ET, _D), jnp.float32),
        pltpu.VMEM((16,), jnp.int32),
        pltpu.SemaphoreType.DMA,
        pltpu.SemaphoreType.DMA,
        pltpu.SemaphoreType.DMA,
    ],
    mesh=_mesh,
    compiler_params=pltpu.CompilerParams(use_tc_tiling_on_sc=False, needs_layout_passes=False),
)(_layer_body)


def _final_body(x0_h, x1_h, x2_h, x3_h, users_h, items_h, gamma_h,
                ubuf, ibuf, usum, isum, tbuf, gbuf, sem):
  c = lax.axis_index("c")
  s = lax.axis_index("s")
  wid = s * _NC + c
  b0 = wid * _BPW
  pltpu.sync_copy(users_h.at[pl.ds(b0, _BPW)], ubuf)
  pltpu.sync_copy(items_h.at[pl.ds(b0, _BPW)], ibuf)
  for j in range(_BPW // 16):
    ibuf[pl.ds(j * 16, 16)] = ibuf[pl.ds(j * 16, 16)] + _N_USER

  pltpu.async_copy(x0_h.at[ubuf], usum, sem).wait()
  pltpu.async_copy(x0_h.at[ibuf], isum, sem).wait()
  for xl_h in (x1_h, x2_h, x3_h):
    pltpu.async_copy(xl_h.at[ubuf], tbuf, sem).wait()

    def accu(i, carry):
      for k in range(_D // 16):
        sl = pl.ds(k * 16, 16)
        usum[i, sl] = usum[i, sl] + tbuf[i, sl]
      return carry
    lax.fori_loop(0, _BPW, accu, 0)

    pltpu.async_copy(xl_h.at[ibuf], tbuf, sem).wait()

    def acci(i, carry):
      for k in range(_D // 16):
        sl = pl.ds(k * 16, 16)
        isum[i, sl] = isum[i, sl] + tbuf[i, sl]
      return carry
    lax.fori_loop(0, _BPW, acci, 0)

  lane = lax.iota(jnp.int32, 16)

  def group(g, carry):
    out16 = jnp.zeros((16,), jnp.float32)
    for l in range(16):
      i = g * 16 + l
      acc2 = jnp.zeros((16,), jnp.float32)
      for k in range(_D // 16):
        sl = pl.ds(k * 16, 16)
        acc2 = acc2 + usum[i, sl] * isum[i, sl]
      dot = jnp.sum(acc2) * jnp.float32(0.0625)
      out16 = jnp.where(lane == l, jnp.full((16,), dot), out16)
    gbuf[pl.ds(g * 16, 16)] = out16
    return carry
  lax.fori_loop(0, _BPW // 16, group, 0)

  pltpu.sync_copy(gbuf, gamma_h.at[pl.ds(b0, _BPW)])


_final = functools.partial(
    pl.kernel,
    out_type=jax.ShapeDtypeStruct((_B,), jnp.float32),
    scratch_types=[
        pltpu.VMEM((_BPW,), jnp.int32),
        pltpu.VMEM((_BPW,), jnp.int32),
        pltpu.VMEM((_BPW, _D), jnp.float32),
        pltpu.VMEM((_BPW, _D), jnp.float32),
        pltpu.VMEM((_BPW, _D), jnp.float32),
        pltpu.VMEM((_BPW,), jnp.float32),
        pltpu.SemaphoreType.DMA,
    ],
    mesh=_mesh,
    compiler_params=pltpu.CompilerParams(use_tc_tiling_on_sc=False, needs_layout_passes=False),
)(_final_body)


def kernel(edge_index, adj_vals, users, items, emb_user, emb_item):
  row = edge_index[0]
  col = edge_index[1]
  x0 = jnp.concatenate([emb_user, emb_item], axis=0)
  ccol, cval, clidx, cnt = _compact(row, col, adj_vals)
  x1 = _layer(ccol, cval, clidx, cnt, x0)
  x2 = _layer(ccol, cval, clidx, cnt, x1)
  x3 = _layer(ccol, cval, clidx, cnt, x2)
  return _final(x0, x1, x2, x3, users, items)


# speedup vs baseline: 7.3396x; 1.1032x over previous
import functools
import jax
import jax.numpy as jnp
from jax import lax
from jax.experimental import pallas as pl
from jax.experimental.pallas import tpu as pltpu
from jax.experimental.pallas import tpu_sc as plsc

_N_USER = 20000
_N_ITEM = 30000
_N = _N_USER + _N_ITEM
_E = 800000
_D = 64
_B = 4096
_NC, _NS = 2, 16
_RPC = 25088
_SPR = _RPC // _NS
_WCH = 16
_EPS = _E // _NS
_EC = 400
_ET = 80
_NT = _EC // _ET
_NCH = _EPS // _EC
_BPW = _B // (_NC * _NS)

_CAP = _EPS + _EC
_SG = 816

_mesh = plsc.VectorSubcoreMesh(core_axis_name="c", subcore_axis_name="s")


def _compact_body(row_h, col_h, val_h, ccol_h, cval_h, clidx_h, cnt_h,
                  rbuf, cbuf, vbuf, scol, sval, slidx, cntv, esem):
  c = lax.axis_index("c")
  s = lax.axis_index("s")
  base = c * _RPC
  e_base = s * _EPS

  def load_group(g, p):
    e0 = e_base + g * _EC
    pltpu.async_copy(row_h.at[pl.ds(e0, _EC)], rbuf.at[p], esem)
    pltpu.async_copy(col_h.at[pl.ds(e0, _EC)], cbuf.at[p], esem)
    pltpu.async_copy(val_h.at[pl.ds(e0, _EC)], vbuf.at[p], esem)

  def wait_group(p):
    pltpu.make_async_copy(row_h.at[pl.ds(0, _EC)], rbuf.at[p], esem).wait()
    pltpu.make_async_copy(col_h.at[pl.ds(0, _EC)], cbuf.at[p], esem).wait()
    pltpu.make_async_copy(val_h.at[pl.ds(0, _EC)], vbuf.at[p], esem).wait()

  load_group(0, 0)
  load_group(1, 1)

  def grp(g, carry):
    w, off = carry
    gp = lax.rem(g, 2)
    wait_group(gp)
    for j in range(_EC // 16):
      r = rbuf[gp, pl.ds(j * 16, 16)]
      li = r - base
      owned = (li >= 0) & (li < _RPC)
      plsc.store_compressed(scol.at[pl.ds(w, 16)],
                            cbuf[gp, pl.ds(j * 16, 16)], mask=owned)
      plsc.store_compressed(sval.at[pl.ds(w, 16)],
                            vbuf[gp, pl.ds(j * 16, 16)], mask=owned)
      plsc.store_compressed(slidx.at[pl.ds(w, 16)], li, mask=owned)
      w = w + plsc.all_reduce_population_count(owned)[0]

    @pl.when(g + 2 < _NCH)
    def _prefetch():
      load_group(g + 2, gp)

    def flush(wo):
      w_, off_ = wo
      off_ = pl.multiple_of(off_, 8)
      pltpu.sync_copy(scol.at[pl.ds(0, _EC)],
                      ccol_h.at[c, s, pl.ds(off_, _EC)])
      pltpu.sync_copy(sval.at[pl.ds(0, _EC)],
                      cval_h.at[c, s, pl.ds(off_, _EC)])
      pltpu.sync_copy(slidx.at[pl.ds(0, _EC)],
                      clidx_h.at[c, s, pl.ds(off_, _EC)])
      for j in range(_EC // 16):
        sl_src = pl.ds(_EC + j * 16, 16)
        sl_dst = pl.ds(j * 16, 16)
        scol[sl_dst] = scol[sl_src]
        sval[sl_dst] = sval[sl_src]
        slidx[sl_dst] = slidx[sl_src]
      return (w_ - _EC, off_ + _EC)

    return lax.cond(w >= _EC, flush, lambda wo: wo, (w, off))

  w, off = lax.fori_loop(0, _NCH, grp, (0, 0))

  lane = lax.iota(jnp.int32, 16)
  for j in range(_EC // 16):
    sl = pl.ds(j * 16, 16)
    m = (j * 16 + lane) < w
    scol[sl] = jnp.where(m, scol[sl], 0)
    sval[sl] = jnp.where(m, sval[sl], jnp.float32(0.0))
    slidx[sl] = jnp.where(m, slidx[sl], 0)
  off = pl.multiple_of(off, 8)
  pltpu.sync_copy(scol.at[pl.ds(0, _EC)], ccol_h.at[c, s, pl.ds(off, _EC)])
  pltpu.sync_copy(sval.at[pl.ds(0, _EC)], cval_h.at[c, s, pl.ds(off, _EC)])
  pltpu.sync_copy(slidx.at[pl.ds(0, _EC)], clidx_h.at[c, s, pl.ds(off, _EC)])
  cntv[pl.ds(0, 16)] = jnp.full((16,), off + _EC)
  pltpu.sync_copy(cntv, cnt_h.at[c, s])


_compact = functools.partial(
    pl.kernel,
    out_type=(
        jax.ShapeDtypeStruct((_NC, _NS, _CAP), jnp.int32),
        jax.ShapeDtypeStruct((_NC, _NS, _CAP), jnp.float32),
        jax.ShapeDtypeStruct((_NC, _NS, _CAP), jnp.int32),
        jax.ShapeDtypeStruct((_NC, _NS, 16), jnp.int32),
    ),
    scratch_types=[
        pltpu.VMEM((2, _EC), jnp.int32),
        pltpu.VMEM((2, _EC), jnp.int32),
        pltpu.VMEM((2, _EC), jnp.float32),
        pltpu.VMEM((_SG,), jnp.int32),
        pltpu.VMEM((_SG,), jnp.float32),
        pltpu.VMEM((_SG,), jnp.int32),
        pltpu.VMEM((16,), jnp.int32),
        pltpu.SemaphoreType.DMA,
    ],
    mesh=_mesh,
    compiler_params=pltpu.CompilerParams(use_tc_tiling_on_sc=False,
                                         needs_layout_passes=False),
)(_compact_body)


def _layer_body(ccol_h, cval_h, clidx_h, cnt_h, xsrc_h, xdst_h,
                acc, cbuf, vbuf, lidx, rows, cntv, gsem, esem, ssem):
  c = lax.axis_index("c")
  s = lax.axis_index("s")
  pltpu.sync_copy(cnt_h.at[c, s], cntv)
  ng = cntv[pl.ds(0, 16)][0] // _EC
  zero16 = jnp.zeros((16,), jnp.float32)

  def zrow(i, carry):
    for k in range(_D // 16):
      rows[0, i, pl.ds(k * 16, 16)] = zero16
    return carry
  lax.fori_loop(0, _WCH, zrow, 0)

  def zacc(i, carry):
    pltpu.async_copy(rows.at[0, pl.ds(0, _WCH)],
                     acc.at[pl.ds(s * _SPR + i * _WCH, _WCH)], esem)
    return carry
  lax.fori_loop(0, _SPR // _WCH, zacc, 0)

  def zwait(i, carry):
    pltpu.make_async_copy(rows.at[0, pl.ds(0, _WCH)],
                          acc.at[pl.ds(s * _SPR, _WCH)], esem).wait()
    return carry
  lax.fori_loop(0, _SPR // _WCH, zwait, 0)

  plsc.subcore_barrier()

  def load_group(g, p):
    e0 = pl.multiple_of(g * _EC, 8)
    pltpu.async_copy(ccol_h.at[c, s, pl.ds(e0, _EC)], cbuf.at[p], esem)
    pltpu.async_copy(cval_h.at[c, s, pl.ds(e0, _EC)], vbuf.at[p], esem)
    for t in range(_NT):
      pltpu.async_copy(clidx_h.at[c, s, pl.ds(e0 + t * _ET, _ET)],
                       lidx.at[p, t], esem)

  def wait_group(p):
    pltpu.make_async_copy(ccol_h.at[c, s, pl.ds(0, _EC)],
                          cbuf.at[p], esem).wait()
    pltpu.make_async_copy(cval_h.at[c, s, pl.ds(0, _EC)],
                          vbuf.at[p], esem).wait()
    for t in range(_NT):
      pltpu.make_async_copy(clidx_h.at[c, s, pl.ds(0, _ET)],
                            lidx.at[p, t], esem).wait()

  def fire_gather(p, t, b):
    pltpu.async_copy(xsrc_h.at[cbuf.at[p, pl.ds(t * _ET, _ET)]],
                     rows.at[b], gsem)

  def wait_gather(b):
    pltpu.make_async_copy(xsrc_h.at[pl.ds(0, _ET)], rows.at[b], gsem).wait()

  def wait_scatter():
    pltpu.make_async_copy(rows.at[0], acc.at[lidx.at[0, 0]], ssem).wait()

  load_group(0, 0)
  wait_group(0)

  @pl.when(ng > 1)
  def _pre():
    load_group(1, 1)
  fire_gather(0, 0, 0)
  fire_gather(0, 1, 1)
  fire_gather(0, 2, 2)

  def group_body(g, carry):
    gp = lax.rem(g, 3)
    gn = g + 1
    gnp = lax.rem(gn, 3)
    for t in range(_NT):
      m = g * _NT + t
      b = lax.rem(m, 5)
      wait_gather(b)
      @pl.when(m >= 2)
      def _drain():
        wait_scatter()
      if t < _NT - 3:
        fire_gather(gp, t + 3, lax.rem(m + 3, 5))
      elif t == _NT - 3:
        @pl.when(gn < ng)
        def _boundary():
          wait_group(gnp)

          @pl.when(gn + 1 < ng)
          def _prefetch():
            load_group(gn + 1, lax.rem(gn + 1, 3))
          fire_gather(gnp, 0, lax.rem(m + 3, 5))
      elif t == _NT - 2:
        @pl.when(gn < ng)
        def _next2():
          fire_gather(gnp, 1, lax.rem(m + 3, 5))
      else:
        @pl.when(gn < ng)
        def _next3():
          fire_gather(gnp, 2, lax.rem(m + 3, 5))
      def scale(j, carry2):
        vv = vbuf[gp, pl.ds(t * _ET + j * 16, 16)]
        for l in range(16):
          e = j * 16 + l
          vsp = lax.gather(
              vv, jnp.full((16, 1), l, jnp.int32),
              lax.GatherDimensionNumbers(offset_dims=(),
                                         collapsed_slice_dims=(0,),
                                         start_index_map=(0,)),
              (1,), mode=lax.GatherScatterMode.PROMISE_IN_BOUNDS)
          for k in range(_D // 16):
            rows[b, e, pl.ds(k * 16, 16)] = (
                rows[b, e, pl.ds(k * 16, 16)] * vsp)
        return carry2
      lax.fori_loop(0, _ET // 16, scale, 0)
      pltpu.async_copy(rows.at[b], acc.at[lidx.at[gp, t]], ssem, add=True)
    return carry
  lax.fori_loop(0, ng, group_body, 0)
  wait_scatter()
  wait_scatter()

  plsc.subcore_barrier()

  g0 = c * _RPC + s * _SPR
  nch = jnp.minimum(_SPR, jnp.maximum(0, _N - g0)) // _WCH

  def wback(i, carry):
    pltpu.async_copy(acc.at[pl.ds(s * _SPR + i * _WCH, _WCH)],
                     xdst_h.at[pl.ds(g0 + i * _WCH, _WCH)], esem)
    return carry
  lax.fori_loop(0, nch, wback, 0)

  def wbwait(i, carry):
    pltpu.make_async_copy(acc.at[pl.ds(s * _SPR, _WCH)],
                          xdst_h.at[pl.ds(g0, _WCH)], esem).wait()
    return carry
  lax.fori_loop(0, nch, wbwait, 0)


_layer = functools.partial(
    pl.kernel,
    out_type=jax.ShapeDtypeStruct((_N, _D), jnp.float32),
    scratch_types=[
        pltpu.VMEM_SHARED((_RPC, _D), jnp.float32),
        pltpu.VMEM((3, _EC), jnp.int32),
        pltpu.VMEM((3, _EC), jnp.float32),
        pltpu.VMEM((3, _NT, _ET), jnp.int32),
        pltpu.VMEM((5, _ET, _D), jnp.float32),
        pltpu.VMEM((16,), jnp.int32),
        pltpu.SemaphoreType.DMA,
        pltpu.SemaphoreType.DMA,
        pltpu.SemaphoreType.DMA,
    ],
    mesh=_mesh,
    compiler_params=pltpu.CompilerParams(use_tc_tiling_on_sc=False, needs_layout_passes=False),
)(_layer_body)


def _final_body(x0_h, x1_h, x2_h, x3_h, users_h, items_h, gamma_h,
                ubuf, ibuf, usum, isum, tbuf, gbuf, sem):
  c = lax.axis_index("c")
  s = lax.axis_index("s")
  wid = s * _NC + c
  b0 = wid * _BPW
  pltpu.sync_copy(users_h.at[pl.ds(b0, _BPW)], ubuf)
  pltpu.sync_copy(items_h.at[pl.ds(b0, _BPW)], ibuf)
  for j in range(_BPW // 16):
    ibuf[pl.ds(j * 16, 16)] = ibuf[pl.ds(j * 16, 16)] + _N_USER

  pltpu.async_copy(x0_h.at[ubuf], usum, sem).wait()
  pltpu.async_copy(x0_h.at[ibuf], isum, sem).wait()
  for xl_h in (x1_h, x2_h, x3_h):
    pltpu.async_copy(xl_h.at[ubuf], tbuf, sem).wait()

    def accu(i, carry):
      for k in range(_D // 16):
        sl = pl.ds(k * 16, 16)
        usum[i, sl] = usum[i, sl] + tbuf[i, sl]
      return carry
    lax.fori_loop(0, _BPW, accu, 0)

    pltpu.async_copy(xl_h.at[ibuf], tbuf, sem).wait()

    def acci(i, carry):
      for k in range(_D // 16):
        sl = pl.ds(k * 16, 16)
        isum[i, sl] = isum[i, sl] + tbuf[i, sl]
      return carry
    lax.fori_loop(0, _BPW, acci, 0)

  lane = lax.iota(jnp.int32, 16)

  def group(g, carry):
    out16 = jnp.zeros((16,), jnp.float32)
    for l in range(16):
      i = g * 16 + l
      acc2 = jnp.zeros((16,), jnp.float32)
      for k in range(_D // 16):
        sl = pl.ds(k * 16, 16)
        acc2 = acc2 + usum[i, sl] * isum[i, sl]
      dot = jnp.sum(acc2) * jnp.float32(0.0625)
      out16 = jnp.where(lane == l, jnp.full((16,), dot), out16)
    gbuf[pl.ds(g * 16, 16)] = out16
    return carry
  lax.fori_loop(0, _BPW // 16, group, 0)

  pltpu.sync_copy(gbuf, gamma_h.at[pl.ds(b0, _BPW)])


_final = functools.partial(
    pl.kernel,
    out_type=jax.ShapeDtypeStruct((_B,), jnp.float32),
    scratch_types=[
        pltpu.VMEM((_BPW,), jnp.int32),
        pltpu.VMEM((_BPW,), jnp.int32),
        pltpu.VMEM((_BPW, _D), jnp.float32),
        pltpu.VMEM((_BPW, _D), jnp.float32),
        pltpu.VMEM((_BPW, _D), jnp.float32),
        pltpu.VMEM((_BPW,), jnp.float32),
        pltpu.SemaphoreType.DMA,
    ],
    mesh=_mesh,
    compiler_params=pltpu.CompilerParams(use_tc_tiling_on_sc=False, needs_layout_passes=False),
)(_final_body)


def kernel(edge_index, adj_vals, users, items, emb_user, emb_item):
  row = edge_index[0]
  col = edge_index[1]
  x0 = jnp.concatenate([emb_user, emb_item], axis=0)
  ccol, cval, clidx, cnt = _compact(row, col, adj_vals)
  x1 = _layer(ccol, cval, clidx, cnt, x0)
  x2 = _layer(ccol, cval, clidx, cnt, x1)
  x3 = _layer(ccol, cval, clidx, cnt, x2)
  return _final(x0, x1, x2, x3, users, items)


# speedup vs baseline: 11.5736x; 1.5769x over previous
import functools
import jax
import jax.numpy as jnp
from jax import lax
from jax.experimental import pallas as pl
from jax.experimental.pallas import tpu as pltpu
from jax.experimental.pallas import tpu_sc as plsc

_N_USER = 20000
_N_ITEM = 30000
_N = _N_USER + _N_ITEM
_E = 800000
_D = 64
_B = 4096
_NC, _NS = 2, 16
_RPC = 25088
_SPR = _RPC // _NS
_WCH = 16
_EPS = _E // _NS
_EC = 400
_ET = 80
_NT = _EC // _ET
_NCH = _EPS // _EC
_BPW = _B // (_NC * _NS)

_CAP = _EPS + _EC
_SG = 816

_mesh = plsc.VectorSubcoreMesh(core_axis_name="c", subcore_axis_name="s")


def _compact_body(row_h, col_h, val_h, ccol_h, cval_h, clidx_h, cnt_h,
                  rbuf, cbuf, vbuf, scol, sval, slidx, cntv, esem):
  c = lax.axis_index("c")
  s = lax.axis_index("s")
  base = c * _RPC
  e_base = s * _EPS

  def load_group(g, p):
    e0 = e_base + g * _EC
    pltpu.async_copy(row_h.at[pl.ds(e0, _EC)], rbuf.at[p], esem)
    pltpu.async_copy(col_h.at[pl.ds(e0, _EC)], cbuf.at[p], esem)
    pltpu.async_copy(val_h.at[pl.ds(e0, _EC)], vbuf.at[p], esem)

  def wait_group(p):
    pltpu.make_async_copy(row_h.at[pl.ds(0, _EC)], rbuf.at[p], esem).wait()
    pltpu.make_async_copy(col_h.at[pl.ds(0, _EC)], cbuf.at[p], esem).wait()
    pltpu.make_async_copy(val_h.at[pl.ds(0, _EC)], vbuf.at[p], esem).wait()

  load_group(0, 0)
  load_group(1, 1)

  def grp(g, carry):
    w, off = carry
    gp = lax.rem(g, 2)
    wait_group(gp)
    for j in range(_EC // 16):
      r = rbuf[gp, pl.ds(j * 16, 16)]
      li = r - base
      owned = (li >= 0) & (li < _RPC)
      plsc.store_compressed(scol.at[pl.ds(w, 16)],
                            cbuf[gp, pl.ds(j * 16, 16)], mask=owned)
      plsc.store_compressed(sval.at[pl.ds(w, 16)],
                            vbuf[gp, pl.ds(j * 16, 16)], mask=owned)
      plsc.store_compressed(slidx.at[pl.ds(w, 16)], li, mask=owned)
      w = w + plsc.all_reduce_population_count(owned)[0]

    @pl.when(g + 2 < _NCH)
    def _prefetch():
      load_group(g + 2, gp)

    def flush(wo):
      w_, off_ = wo
      off_ = pl.multiple_of(off_, 8)
      pltpu.sync_copy(scol.at[pl.ds(0, _EC)],
                      ccol_h.at[c, s, pl.ds(off_, _EC)])
      pltpu.sync_copy(sval.at[pl.ds(0, _EC)],
                      cval_h.at[c, s, pl.ds(off_, _EC)])
      pltpu.sync_copy(slidx.at[pl.ds(0, _EC)],
                      clidx_h.at[c, s, pl.ds(off_, _EC)])
      for j in range(_EC // 16):
        sl_src = pl.ds(_EC + j * 16, 16)
        sl_dst = pl.ds(j * 16, 16)
        scol[sl_dst] = scol[sl_src]
        sval[sl_dst] = sval[sl_src]
        slidx[sl_dst] = slidx[sl_src]
      return (w_ - _EC, off_ + _EC)

    return lax.cond(w >= _EC, flush, lambda wo: wo, (w, off))

  w, off = lax.fori_loop(0, _NCH, grp, (0, 0))

  lane = lax.iota(jnp.int32, 16)
  for j in range(_EC // 16):
    sl = pl.ds(j * 16, 16)
    m = (j * 16 + lane) < w
    scol[sl] = jnp.where(m, scol[sl], 0)
    sval[sl] = jnp.where(m, sval[sl], jnp.float32(0.0))
    slidx[sl] = jnp.where(m, slidx[sl], 0)
  off = pl.multiple_of(off, 8)
  pltpu.sync_copy(scol.at[pl.ds(0, _EC)], ccol_h.at[c, s, pl.ds(off, _EC)])
  pltpu.sync_copy(sval.at[pl.ds(0, _EC)], cval_h.at[c, s, pl.ds(off, _EC)])
  pltpu.sync_copy(slidx.at[pl.ds(0, _EC)], clidx_h.at[c, s, pl.ds(off, _EC)])
  cntv[pl.ds(0, 16)] = jnp.full((16,), off + _EC)
  pltpu.sync_copy(cntv, cnt_h.at[c, s])


_compact = functools.partial(
    pl.kernel,
    out_type=(
        jax.ShapeDtypeStruct((_NC, _NS, _CAP), jnp.int32),
        jax.ShapeDtypeStruct((_NC, _NS, _CAP), jnp.float32),
        jax.ShapeDtypeStruct((_NC, _NS, _CAP), jnp.int32),
        jax.ShapeDtypeStruct((_NC, _NS, 16), jnp.int32),
    ),
    scratch_types=[
        pltpu.VMEM((2, _EC), jnp.int32),
        pltpu.VMEM((2, _EC), jnp.int32),
        pltpu.VMEM((2, _EC), jnp.float32),
        pltpu.VMEM((_SG,), jnp.int32),
        pltpu.VMEM((_SG,), jnp.float32),
        pltpu.VMEM((_SG,), jnp.int32),
        pltpu.VMEM((16,), jnp.int32),
        pltpu.SemaphoreType.DMA,
    ],
    mesh=_mesh,
    compiler_params=pltpu.CompilerParams(use_tc_tiling_on_sc=False,
                                         needs_layout_passes=False),
)(_compact_body)


def _layer_body(ccol_h, cval_h, clidx_h, cnt_h, xsrc_h, xdst_h,
                acc, cbuf, vbuf, lidx, rows, cntv, gsem, esem, ssem):
  c = lax.axis_index("c")
  s = lax.axis_index("s")
  pltpu.sync_copy(cnt_h.at[c, s], cntv)
  ng = cntv[pl.ds(0, 16)][0] // _EC
  zero16 = jnp.zeros((16,), jnp.float32)

  def zrow(i, carry):
    for k in range(_D // 16):
      rows[0, i, pl.ds(k * 16, 16)] = zero16
    return carry
  lax.fori_loop(0, _WCH, zrow, 0)

  def zacc(i, carry):
    pltpu.async_copy(rows.at[0, pl.ds(0, _WCH)],
                     acc.at[pl.ds(s * _SPR + i * _WCH, _WCH)], esem)
    return carry
  lax.fori_loop(0, _SPR // _WCH, zacc, 0)

  def zwait(i, carry):
    pltpu.make_async_copy(rows.at[0, pl.ds(0, _WCH)],
                          acc.at[pl.ds(s * _SPR, _WCH)], esem).wait()
    return carry
  lax.fori_loop(0, _SPR // _WCH, zwait, 0)

  plsc.subcore_barrier()

  def load_group(g, p):
    e0 = pl.multiple_of(g * _EC, 8)
    pltpu.async_copy(ccol_h.at[c, s, pl.ds(e0, _EC)], cbuf.at[p], esem)
    pltpu.async_copy(cval_h.at[c, s, pl.ds(e0, _EC)], vbuf.at[p], esem)
    for t in range(_NT):
      pltpu.async_copy(clidx_h.at[c, s, pl.ds(e0 + t * _ET, _ET)],
                       lidx.at[p, t], esem)

  def wait_group(p):
    pltpu.make_async_copy(ccol_h.at[c, s, pl.ds(0, _EC)],
                          cbuf.at[p], esem).wait()
    pltpu.make_async_copy(cval_h.at[c, s, pl.ds(0, _EC)],
                          vbuf.at[p], esem).wait()
    for t in range(_NT):
      pltpu.make_async_copy(clidx_h.at[c, s, pl.ds(0, _ET)],
                            lidx.at[p, t], esem).wait()

  def fire_gather(p, t, b):
    pltpu.async_copy(xsrc_h.at[cbuf.at[p, pl.ds(t * _ET, _ET)]],
                     rows.at[b], gsem)

  def wait_gather(b):
    pltpu.make_async_copy(xsrc_h.at[pl.ds(0, _ET)], rows.at[b], gsem).wait()

  def wait_scatter():
    pltpu.make_async_copy(rows.at[0], acc.at[lidx.at[0, 0]], ssem).wait()

  load_group(0, 0)
  wait_group(0)

  @pl.when(ng > 1)
  def _pre():
    load_group(1, 1)
  fire_gather(0, 0, 0)
  fire_gather(0, 1, 1)
  fire_gather(0, 2, 2)

  def group_body(g, carry):
    gp = lax.rem(g, 3)
    gn = g + 1
    gnp = lax.rem(gn, 3)
    for t in range(_NT):
      m = g * _NT + t
      b = lax.rem(m, 5)
      wait_gather(b)
      @pl.when(m >= 2)
      def _drain():
        wait_scatter()
      if t < _NT - 3:
        fire_gather(gp, t + 3, lax.rem(m + 3, 5))
      elif t == _NT - 3:
        @pl.when(gn < ng)
        def _boundary():
          wait_group(gnp)

          @pl.when(gn + 1 < ng)
          def _prefetch():
            load_group(gn + 1, lax.rem(gn + 1, 3))
          fire_gather(gnp, 0, lax.rem(m + 3, 5))
      elif t == _NT - 2:
        @pl.when(gn < ng)
        def _next2():
          fire_gather(gnp, 1, lax.rem(m + 3, 5))
      else:
        @pl.when(gn < ng)
        def _next3():
          fire_gather(gnp, 2, lax.rem(m + 3, 5))
      for j in range(_ET // 16):
        vv = vbuf[gp, pl.ds(t * _ET + j * 16, 16)]
        for l in range(16):
          e = j * 16 + l
          vsp = lax.gather(
              vv, jnp.full((16, 1), l, jnp.int32),
              lax.GatherDimensionNumbers(offset_dims=(),
                                         collapsed_slice_dims=(0,),
                                         start_index_map=(0,)),
              (1,), mode=lax.GatherScatterMode.PROMISE_IN_BOUNDS)
          for k in range(_D // 16):
            rows[b, e, pl.ds(k * 16, 16)] = (
                rows[b, e, pl.ds(k * 16, 16)] * vsp)
      pltpu.async_copy(rows.at[b], acc.at[lidx.at[gp, t]], ssem, add=True)
    return carry
  lax.fori_loop(0, ng, group_body, 0)
  wait_scatter()
  wait_scatter()

  plsc.subcore_barrier()

  g0 = c * _RPC + s * _SPR
  nch = jnp.minimum(_SPR, jnp.maximum(0, _N - g0)) // _WCH

  def wback(i, carry):
    pltpu.async_copy(acc.at[pl.ds(s * _SPR + i * _WCH, _WCH)],
                     xdst_h.at[pl.ds(g0 + i * _WCH, _WCH)], esem)
    return carry
  lax.fori_loop(0, nch, wback, 0)

  def wbwait(i, carry):
    pltpu.make_async_copy(acc.at[pl.ds(s * _SPR, _WCH)],
                          xdst_h.at[pl.ds(g0, _WCH)], esem).wait()
    return carry
  lax.fori_loop(0, nch, wbwait, 0)


_layer = functools.partial(
    pl.kernel,
    out_type=jax.ShapeDtypeStruct((_N, _D), jnp.float32),
    scratch_types=[
        pltpu.VMEM_SHARED((_RPC, _D), jnp.float32),
        pltpu.VMEM((3, _EC), jnp.int32),
        pltpu.VMEM((3, _EC), jnp.float32),
        pltpu.VMEM((3, _NT, _ET), jnp.int32),
        pltpu.VMEM((5, _ET, _D), jnp.float32),
        pltpu.VMEM((16,), jnp.int32),
        pltpu.SemaphoreType.DMA,
        pltpu.SemaphoreType.DMA,
        pltpu.SemaphoreType.DMA,
    ],
    mesh=_mesh,
    compiler_params=pltpu.CompilerParams(use_tc_tiling_on_sc=False, needs_layout_passes=False),
)(_layer_body)


def _final_body(x0_h, x1_h, x2_h, x3_h, users_h, items_h, gamma_h,
                ubuf, ibuf, usum, isum, tbuf, gbuf, sem):
  c = lax.axis_index("c")
  s = lax.axis_index("s")
  wid = s * _NC + c
  b0 = wid * _BPW
  pltpu.sync_copy(users_h.at[pl.ds(b0, _BPW)], ubuf)
  pltpu.sync_copy(items_h.at[pl.ds(b0, _BPW)], ibuf)
  for j in range(_BPW // 16):
    ibuf[pl.ds(j * 16, 16)] = ibuf[pl.ds(j * 16, 16)] + _N_USER

  pltpu.async_copy(x0_h.at[ubuf], usum, sem).wait()
  pltpu.async_copy(x0_h.at[ibuf], isum, sem).wait()
  for xl_h in (x1_h, x2_h, x3_h):
    pltpu.async_copy(xl_h.at[ubuf], tbuf, sem).wait()

    def accu(i, carry):
      for k in range(_D // 16):
        sl = pl.ds(k * 16, 16)
        usum[i, sl] = usum[i, sl] + tbuf[i, sl]
      return carry
    lax.fori_loop(0, _BPW, accu, 0)

    pltpu.async_copy(xl_h.at[ibuf], tbuf, sem).wait()

    def acci(i, carry):
      for k in range(_D // 16):
        sl = pl.ds(k * 16, 16)
        isum[i, sl] = isum[i, sl] + tbuf[i, sl]
      return carry
    lax.fori_loop(0, _BPW, acci, 0)

  lane = lax.iota(jnp.int32, 16)

  def group(g, carry):
    out16 = jnp.zeros((16,), jnp.float32)
    for l in range(16):
      i = g * 16 + l
      acc2 = jnp.zeros((16,), jnp.float32)
      for k in range(_D // 16):
        sl = pl.ds(k * 16, 16)
        acc2 = acc2 + usum[i, sl] * isum[i, sl]
      dot = jnp.sum(acc2) * jnp.float32(0.0625)
      out16 = jnp.where(lane == l, jnp.full((16,), dot), out16)
    gbuf[pl.ds(g * 16, 16)] = out16
    return carry
  lax.fori_loop(0, _BPW // 16, group, 0)

  pltpu.sync_copy(gbuf, gamma_h.at[pl.ds(b0, _BPW)])


_final = functools.partial(
    pl.kernel,
    out_type=jax.ShapeDtypeStruct((_B,), jnp.float32),
    scratch_types=[
        pltpu.VMEM((_BPW,), jnp.int32),
        pltpu.VMEM((_BPW,), jnp.int32),
        pltpu.VMEM((_BPW, _D), jnp.float32),
        pltpu.VMEM((_BPW, _D), jnp.float32),
        pltpu.VMEM((_BPW, _D), jnp.float32),
        pltpu.VMEM((_BPW,), jnp.float32),
        pltpu.SemaphoreType.DMA,
    ],
    mesh=_mesh,
    compiler_params=pltpu.CompilerParams(use_tc_tiling_on_sc=False, needs_layout_passes=False),
)(_final_body)


def kernel(edge_index, adj_vals, users, items, emb_user, emb_item):
  row = edge_index[0]
  col = edge_index[1]
  x0 = jnp.concatenate([emb_user, emb_item], axis=0)
  ccol, cval, clidx, cnt = _compact(row, col, adj_vals)
  x1 = _layer(ccol, cval, clidx, cnt, x0)
  x2 = _layer(ccol, cval, clidx, cnt, x1)
  x3 = _layer(ccol, cval, clidx, cnt, x2)
  return _final(x0, x1, x2, x3, users, items)
